# Initial kernel scaffold; baseline (speedup 1.0000x reference)
import jax
import jax.numpy as jnp
from jax.experimental import pallas as pl


def kernel(x, pe):
    raise NotImplementedError("write your pallas kernel here")


# speedup vs baseline: 2.3703x; 2.3703x over previous
import functools

import jax
import jax.numpy as jnp
from jax import lax
from jax.experimental import pallas as pl
from jax.experimental.pallas import tpu as pltpu
from jax.experimental.pallas import tpu_sc as plsc


@functools.lru_cache(maxsize=None)
def _make_gather(n_workers, num_cores, n_chunks, chunk, d_model, n_rows):
    n_total = n_workers * n_chunks * chunk
    mesh = plsc.VectorSubcoreMesh(core_axis_name="c", subcore_axis_name="s")

    @functools.partial(
        pl.kernel,
        mesh=mesh,
        out_type=jax.ShapeDtypeStruct((n_total, d_model), jnp.float32),
        scratch_types=[
            pltpu.VMEM((n_chunks, chunk), jnp.int32),
            pltpu.VMEM((2, chunk, d_model), jnp.float32),
            pltpu.SemaphoreType.DMA,
            pltpu.SemaphoreType.DMA,
        ],
    )
    def gather_kernel(idx_hbm, table_hbm, out_hbm, idx_v, rows_v, sem0, sem1):
        wid = lax.axis_index("s") * num_cores + lax.axis_index("c")
        base = wid * (n_chunks * chunk)
        sems = (sem0, sem1)

        pltpu.sync_copy(idx_hbm.at[wid], idx_v)

        def start_gather(g, b):
            pltpu.async_copy(table_hbm.at[idx_v.at[g]], rows_v.at[b], sems[b])

        def finish_chunk(g, b):
            pltpu.make_async_copy(
                table_hbm.at[idx_v.at[g]], rows_v.at[b], sems[b]
            ).wait()
            pltpu.sync_copy(
                rows_v.at[b], out_hbm.at[pl.ds(base + g * chunk, chunk)]
            )

        start_gather(0, 0)
        start_gather(1, 1)

        def body(g2, carry):
            for b in range(2):
                g = g2 * 2 + b
                finish_chunk(g, b)
                start_gather(g + 2, b)
            return carry

        lax.fori_loop(0, n_chunks // 2 - 1, body, 0)

        for b in range(2):
            finish_chunk(n_chunks - 2 + b, b)

    return gather_kernel


def kernel(x, pe):
    info = plsc.get_sparse_core_info()
    n_workers = info.num_cores * info.num_subcores
    n_total = x.shape[0] * x.shape[1]
    chunk = 32
    n_chunks = n_total // (n_workers * chunk)
    idx = x.reshape(n_workers, n_chunks, chunk).astype(jnp.int32)
    gather = _make_gather(
        n_workers, info.num_cores, n_chunks, chunk, pe.shape[1], pe.shape[0]
    )
    out = gather(idx, pe)
    return out.reshape(x.shape[0], x.shape[1], pe.shape[1])
